# baseline (speedup 1.0000x reference)
import functools

import jax
import jax.numpy as jnp
from jax import lax
from jax.experimental import pallas as pl
from jax.experimental.pallas import tpu as pltpu
from jax.experimental.pallas import tpu_sc as plsc


@functools.cache
def _build_gather(B, S, V, D, NC, NS):
    NW = NC * NS
    CB = 512
    NBLK = CB // 128
    QB = B // CB
    n_chunks = (S * QB) // NW
    TD = D // 8
    assert B % CB == 0 and (S * QB) % NW == 0 and D % 8 == 0
    assert n_chunks % 2 == 0 and NBLK % 2 == 0

    mesh = plsc.VectorSubcoreMesh(core_axis_name="c", subcore_axis_name="s")

    @functools.partial(
        pl.kernel,
        out_type=jax.ShapeDtypeStruct((S * TD, B // 128, 8, 128), jnp.float32),
        mesh=mesh,
        scratch_types=[
            [pltpu.VMEM((CB,), jnp.int32) for _ in range(2)],
            [pltpu.VMEM((CB, D), jnp.float32) for _ in range(2)],
            [pltpu.VMEM((D, 128), jnp.float32) for _ in range(2)],
            [pltpu.SemaphoreType.DMA for _ in range(2)],
            [pltpu.SemaphoreType.DMA for _ in range(2)],
            pltpu.SemaphoreType.DMA,
        ],
        compiler_params=pltpu.CompilerParams(
            use_tc_tiling_on_sc=False, needs_layout_passes=False),
    )
    def gather_kernel(x_hbm, table_hbm, out_hbm, idxs, rows, tbufs, sis, sgs, so):
        wid = lax.axis_index("s") * NC + lax.axis_index("c")
        base_chunk = wid * n_chunks

        def x_off(chunk):
            return (chunk // QB) * B + (chunk % QB) * CB

        def start_idx(chunk, bb):
            pltpu.async_copy(x_hbm.at[pl.ds(x_off(chunk), CB)], idxs[bb], sis[bb])

        def start_gather(bb):
            pltpu.async_copy(table_hbm.at[idxs[bb]], rows[bb], sgs[bb])

        def wait_idx(bb):
            pltpu.make_async_copy(x_hbm.at[pl.ds(0, CB)], idxs[bb], sis[bb]).wait()

        def wait_gather(bb):
            pltpu.make_async_copy(table_hbm.at[pl.ds(0, CB)], rows[bb], sgs[bb]).wait()

        def wait_out():
            pltpu.make_async_copy(
                tbufs[0].at[pl.ds(0, 8)], out_hbm.at[0, 0], so).wait()

        start_idx(base_chunk, 0)
        wait_idx(0)
        start_gather(0)
        start_idx(base_chunk + 1, 1)

        lanes = lax.broadcasted_iota(jnp.int32, (16,), 0)

        def do_chunk(bb, g):
            chunk = base_chunk + g
            s = chunk // QB
            q = chunk % QB
            wait_gather(bb)
            return chunk, s, q

        def transpose_chunk(bb, s, q):
            def do_block_pair(k2, carry2):
                for kk in range(2):
                    k = k2 * 2 + kk
                    tb = q * NBLK + k
                    tbuf = tbufs[kk]

                    @pl.when(k2 > 0)
                    def _():
                        for _td in range(TD):
                            wait_out()

                    def do_j(j, carry3):
                        row_idx = k * 128 + j * 16 + lanes
                        for d in range(D):
                            col = jnp.full((16,), d, jnp.int32)
                            v = plsc.load_gather(rows[bb], [row_idx, col])
                            tbuf[d, pl.ds(j * 16, 16)] = v
                        return carry3

                    lax.fori_loop(0, 8, do_j, 0)

                    for td in range(TD):
                        pltpu.async_copy(
                            tbuf.at[pl.ds(td * 8, 8)],
                            out_hbm.at[s * TD + td, tb], so)
                return carry2

            lax.fori_loop(0, NBLK // 2, do_block_pair, 0)
            for _ in range(2 * TD):
                wait_out()

        def do_pair(g2, carry):
            for bb in range(2):
                g = 2 * g2 + bb
                chunk, s, q = do_chunk(bb, g)

                if bb == 0:
                    wait_idx(1)
                    start_gather(1)
                else:
                    @pl.when(g2 < (n_chunks // 2) - 1)
                    def _():
                        wait_idx(0)
                        start_gather(0)

                @pl.when(g2 < (n_chunks // 2) - 1)
                def _():
                    start_idx(chunk + 2, bb)

                transpose_chunk(bb, s, q)
            return carry

        lax.fori_loop(0, n_chunks // 2, do_pair, 0)

    return gather_kernel


def kernel(x, table):
    B, S = x.shape
    V, D = table.shape
    info = plsc.get_sparse_core_info()
    f = _build_gather(B, S, V, D, info.num_cores, info.num_subcores)
    xt_flat = jnp.transpose(x).reshape(B * S)
    out4 = f(xt_flat, table)
    TD = D // 8
    out5 = out4.reshape(S, TD, B // 128, 8, 128)
    return out5.transpose(2, 4, 0, 1, 3).reshape(B, S, D)


# speedup vs baseline: 1.6133x; 1.6133x over previous
import functools

import jax
import jax.numpy as jnp
from jax import lax
from jax.experimental import pallas as pl
from jax.experimental.pallas import tpu as pltpu
from jax.experimental.pallas import tpu_sc as plsc


@functools.cache
def _build_gather(B, S, V, D, NC, NS):
    NW = NC * NS
    CB = 512
    NBLK = CB // 128
    QB = B // CB
    n_chunks = (S * QB) // NW
    TD = D // 8
    assert B % CB == 0 and (S * QB) % NW == 0 and D % 8 == 0
    assert n_chunks % 2 == 0 and NBLK % 2 == 0

    mesh = plsc.VectorSubcoreMesh(core_axis_name="c", subcore_axis_name="s")

    @functools.partial(
        pl.kernel,
        out_type=jax.ShapeDtypeStruct((S * TD, B // 128, 8, 128), jnp.float32),
        mesh=mesh,
        scratch_types=[
            [pltpu.VMEM((CB,), jnp.int32) for _ in range(2)],
            [pltpu.VMEM((CB, D), jnp.float32) for _ in range(2)],
            [pltpu.VMEM((D, 129), jnp.float32) for _ in range(2)],
            [pltpu.SemaphoreType.DMA for _ in range(2)],
            [pltpu.SemaphoreType.DMA for _ in range(2)],
            pltpu.SemaphoreType.DMA,
        ],
        compiler_params=pltpu.CompilerParams(
            use_tc_tiling_on_sc=False, needs_layout_passes=False),
    )
    def gather_kernel(x_hbm, table_hbm, out_hbm, idxs, rows, tbufs, sis, sgs, so):
        wid = lax.axis_index("s") * NC + lax.axis_index("c")
        base_chunk = wid * n_chunks

        def x_off(chunk):
            return (chunk // QB) * B + (chunk % QB) * CB

        def start_idx(chunk, bb):
            pltpu.async_copy(x_hbm.at[pl.ds(x_off(chunk), CB)], idxs[bb], sis[bb])

        def start_gather(bb):
            pltpu.async_copy(table_hbm.at[idxs[bb]], rows[bb], sgs[bb])

        def wait_idx(bb):
            pltpu.make_async_copy(x_hbm.at[pl.ds(0, CB)], idxs[bb], sis[bb]).wait()

        def wait_gather(bb):
            pltpu.make_async_copy(table_hbm.at[pl.ds(0, CB)], rows[bb], sgs[bb]).wait()

        def wait_out():
            pltpu.make_async_copy(
                tbufs[0].at[pl.ds(0, 8), pl.ds(0, 128)],
                out_hbm.at[0, 0], so).wait()

        start_idx(base_chunk, 0)
        wait_idx(0)
        start_gather(0)
        start_idx(base_chunk + 1, 1)

        lanes = lax.broadcasted_iota(jnp.int32, (16,), 0)
        d_rows = [lanes + 16 * h for h in range(D // 16)]

        def do_chunk(bb, g):
            chunk = base_chunk + g
            s = chunk // QB
            q = chunk % QB
            wait_gather(bb)
            return chunk, s, q

        def transpose_chunk(bb, s, q):
            def do_block_pair(k2, carry2):
                for kk in range(2):
                    k = k2 * 2 + kk
                    tb = q * NBLK + k
                    tbuf = tbufs[kk]

                    @pl.when(k2 > 0)
                    def _():
                        for _td in range(TD):
                            wait_out()

                    def do_r(r8, carry3):
                        for rr in range(8):
                            r = r8 * 8 + rr
                            row = k * 128 + r
                            colv = jnp.zeros((16,), jnp.int32) + r
                            for h in range(D // 16):
                                v = rows[bb][row, pl.ds(16 * h, 16)]
                                plsc.store_scatter(tbuf, [d_rows[h], colv], v)
                        return carry3

                    lax.fori_loop(0, 16, do_r, 0)

                    for td in range(TD):
                        pltpu.async_copy(
                            tbuf.at[pl.ds(td * 8, 8), pl.ds(0, 128)],
                            out_hbm.at[s * TD + td, tb], so)
                return carry2

            lax.fori_loop(0, NBLK // 2, do_block_pair, 0)
            for _ in range(2 * TD):
                wait_out()

        def do_pair(g2, carry):
            for bb in range(2):
                g = 2 * g2 + bb
                chunk, s, q = do_chunk(bb, g)

                if bb == 0:
                    wait_idx(1)
                    start_gather(1)
                else:
                    @pl.when(g2 < (n_chunks // 2) - 1)
                    def _():
                        wait_idx(0)
                        start_gather(0)

                @pl.when(g2 < (n_chunks // 2) - 1)
                def _():
                    start_idx(chunk + 2, bb)

                transpose_chunk(bb, s, q)
            return carry

        lax.fori_loop(0, n_chunks // 2, do_pair, 0)

    return gather_kernel


def kernel(x, table):
    B, S = x.shape
    V, D = table.shape
    info = plsc.get_sparse_core_info()
    f = _build_gather(B, S, V, D, info.num_cores, info.num_subcores)
    xt_flat = jnp.transpose(x).reshape(B * S)
    out4 = f(xt_flat, table)
    TD = D // 8
    out5 = out4.reshape(S, TD, B // 128, 8, 128)
    return out5.transpose(2, 4, 0, 1, 3).reshape(B, S, D)
